# Initial kernel scaffold; baseline (speedup 1.0000x reference)
import jax
import jax.numpy as jnp
from jax.experimental import pallas as pl


def kernel(pix_to_face, bary_coords, faces, verts):
    raise NotImplementedError("write your pallas kernel here")


# speedup vs baseline: 4.1061x; 4.1061x over previous
import functools

import jax
import jax.numpy as jnp
from jax import lax
from jax.experimental import pallas as pl
from jax.experimental.pallas import tpu as pltpu
from jax.experimental.pallas import tpu_sc as plsc

NC = 2
NS = 16
NW = NC * NS
L = 16
CS = 2048
TW = 8


def _tile_body(n_per_tile, pix_hbm, bary_hbm, faces_hbm, verts_hbm,
               res_hbm, vf_hbm, pix_v, bary_v, vf_v, vfidx_v, rows_v, res_v,
               sem0, sem1):
    wid = lax.axis_index("s") * NC + lax.axis_index("c")
    nchunks = n_per_tile // CS
    iota = lax.iota(jnp.int32, L)
    ones = jnp.full((L,), 1.0, jnp.float32)
    zeros = jnp.full((L,), 0.0, jnp.float32)
    neg1 = jnp.full((L,), -1, jnp.int32)
    splats = [jnp.full((L,), k, jnp.int32) for k in range(4)]
    iota3k = [3 * iota + k for k in range(3)]
    iota4 = 4 * iota

    def chunk_body(ch, carry):
        base = wid * n_per_tile + ch * CS
        pltpu.sync_copy(pix_hbm.at[pl.ds(base, CS)], pix_v)
        faces_cp = pltpu.async_copy(faces_hbm.at[pix_v], vf_v, sem0)
        bary_cp = pltpu.async_copy(bary_hbm.at[pl.ds(3 * base, 3 * CS)],
                                   bary_v, sem1)
        faces_cp.wait()

        def rp(g, c2):
            p = g * L + iota
            for k in range(3):
                vk = plsc.load_gather(vf_v, [p, splats[k]])
                plsc.store_scatter(vfidx_v, [3 * g * L + iota3k[k]], vk)
            return c2

        lax.fori_loop(0, CS // L, rp, 0)
        verts_cp = pltpu.async_copy(verts_hbm.at[vfidx_v], rows_v, sem0)
        pltpu.sync_copy(vfidx_v, vf_hbm.at[pl.ds(3 * base, 3 * CS)])
        bary_cp.wait()
        verts_cp.wait()

        def grp(g, c2):
            p3 = 3 * g * L
            p4 = 4 * g * L
            pixl = pix_v[pl.ds(g * L, L)]
            alpha = jnp.where(pixl != neg1, ones, zeros)
            b0 = plsc.load_gather(bary_v, [p3 + iota3k[0]])
            b1 = plsc.load_gather(bary_v, [p3 + iota3k[1]])
            b2 = plsc.load_gather(bary_v, [p3 + iota3k[2]])
            for c in range(3):
                r0 = plsc.load_gather(rows_v, [p3 + iota3k[0], splats[c]])
                r1 = plsc.load_gather(rows_v, [p3 + iota3k[1], splats[c]])
                r2 = plsc.load_gather(rows_v, [p3 + iota3k[2], splats[c]])
                out_c = b0 * r0 + b1 * r1 + b2 * r2
                plsc.store_scatter(res_v, [p4 + iota4 + c], out_c)
            plsc.store_scatter(res_v, [p4 + iota4 + 3], alpha)
            return c2

        lax.fori_loop(0, CS // L, grp, 0)
        pltpu.sync_copy(res_v, res_hbm.at[pl.ds(4 * base, 4 * CS)])
        return carry

    lax.fori_loop(0, nchunks, chunk_body, 0)


@functools.partial(jax.jit, static_argnames=("n",))
def _sc_call(pix, bary, faces8, verts8, n):
    n_per_tile = n // NW
    mesh = plsc.VectorSubcoreMesh(core_axis_name="c", subcore_axis_name="s",
                                  num_cores=NC, num_subcores=NS)
    kern = pl.kernel(
        functools.partial(_tile_body, n_per_tile),
        out_type=(
            jax.ShapeDtypeStruct((n * 4,), jnp.float32),
            jax.ShapeDtypeStruct((n * 3,), jnp.int32),
        ),
        mesh=mesh,
        compiler_params=pltpu.CompilerParams(needs_layout_passes=False,
                                             use_tc_tiling_on_sc=False),
        scratch_types=[
            pltpu.VMEM((CS,), jnp.int32),
            pltpu.VMEM((3 * CS,), jnp.float32),
            pltpu.VMEM((CS, TW), jnp.int32),
            pltpu.VMEM((3 * CS,), jnp.int32),
            pltpu.VMEM((3 * CS, TW), jnp.float32),
            pltpu.VMEM((4 * CS,), jnp.float32),
            pltpu.SemaphoreType.DMA,
            pltpu.SemaphoreType.DMA,
        ],
    )
    return kern(pix, bary, faces8, verts8)


def kernel(pix_to_face, bary_coords, faces, verts):
    b, h, w = pix_to_face.shape[:3]
    n = b * h * w
    pix = pix_to_face.reshape(n).astype(jnp.int32)
    bary_in = bary_coords.reshape(n * 3)
    faces8 = jnp.pad(faces.astype(jnp.int32), ((0, 0), (0, TW - 3)))
    verts8 = jnp.pad(verts, ((0, 0), (0, TW - 3)))
    res, vf = _sc_call(pix, bary_in, faces8, verts8, n)
    results = res.reshape(b, h, w, 4)
    vertex_faces = vf.reshape(b, h, w, 3)
    bary = jnp.squeeze(bary_coords, axis=-2)
    return results, vertex_faces, bary


# speedup vs baseline: 4.1078x; 1.0004x over previous
import functools

import jax
import jax.numpy as jnp
from jax import lax
from jax.experimental import pallas as pl
from jax.experimental.pallas import tpu as pltpu
from jax.experimental.pallas import tpu_sc as plsc

NC = 2
NS = 16
NW = NC * NS
L = 16
CS = 2048
TW = 8


def _tile_body(n_per_tile, pix_hbm, bary_hbm, faces_hbm, verts_hbm,
               res_hbm, vf_hbm, pix_v, bary_v, vf_v, vfidx_v, rows_v, res_v,
               sem0, sem1):
    wid = lax.axis_index("s") * NC + lax.axis_index("c")
    nchunks = n_per_tile // CS
    iota = lax.iota(jnp.int32, L)
    ones = jnp.full((L,), 1.0, jnp.float32)
    zeros = jnp.full((L,), 0.0, jnp.float32)
    neg1 = jnp.full((L,), -1, jnp.int32)
    splats = [jnp.full((L,), k, jnp.int32) for k in range(4)]
    iota3k = [3 * iota + k for k in range(3)]
    iota4 = 4 * iota

    def chunk_body(ch, carry):
        base = wid * n_per_tile + ch * CS
        pltpu.sync_copy(pix_hbm.at[pl.ds(base, CS)], pix_v)
        faces_cp = pltpu.async_copy(faces_hbm.at[pix_v], vf_v, sem0)
        bary_cp = pltpu.async_copy(bary_hbm.at[pl.ds(3 * base, 3 * CS)],
                                   bary_v, sem1)
        faces_cp.wait()

        def rp(g, c2):
            p = g * L + iota
            for k in range(3):
                vk = plsc.load_gather(vf_v, [p, splats[k]])
                plsc.store_scatter(vfidx_v, [3 * g * L + iota3k[k]], vk)
            return c2

        lax.fori_loop(0, CS // L, rp, 0)
        verts_cp = pltpu.async_copy(verts_hbm.at[vfidx_v], rows_v, sem0)
        pltpu.sync_copy(vfidx_v, vf_hbm.at[pl.ds(3 * base, 3 * CS)])
        bary_cp.wait()
        verts_cp.wait()

        def grp(g, c2):
            p3 = 3 * g * L
            p4 = 4 * g * L
            pixl = pix_v[pl.ds(g * L, L)]
            alpha = jnp.where(pixl != neg1, ones, zeros)
            b0 = plsc.load_gather(bary_v, [p3 + iota3k[0]])
            b1 = plsc.load_gather(bary_v, [p3 + iota3k[1]])
            b2 = plsc.load_gather(bary_v, [p3 + iota3k[2]])
            for c in range(3):
                r0 = plsc.load_gather(rows_v, [p3 + iota3k[0], splats[c]])
                r1 = plsc.load_gather(rows_v, [p3 + iota3k[1], splats[c]])
                r2 = plsc.load_gather(rows_v, [p3 + iota3k[2], splats[c]])
                out_c = b0 * r0 + b1 * r1 + b2 * r2
                plsc.store_scatter(res_v, [p4 + iota4 + c], out_c)
            plsc.store_scatter(res_v, [p4 + iota4 + 3], alpha)
            return c2

        lax.fori_loop(0, CS // L, grp, 0)
        pltpu.sync_copy(res_v, res_hbm.at[pl.ds(4 * base, 4 * CS)])
        return carry

    lax.fori_loop(0, nchunks, chunk_body, 0)


@functools.partial(jax.jit, static_argnames=("n",))
def _sc_call(pix, bary, faces8, verts8, n):
    n_per_tile = n // NW
    mesh = plsc.VectorSubcoreMesh(core_axis_name="c", subcore_axis_name="s",
                                  num_cores=NC, num_subcores=NS)
    kern = pl.kernel(
        functools.partial(_tile_body, n_per_tile),
        out_type=(
            jax.ShapeDtypeStruct((n * 4,), jnp.float32),
            jax.ShapeDtypeStruct((n * 3,), jnp.int32),
        ),
        mesh=mesh,
        compiler_params=pltpu.CompilerParams(needs_layout_passes=False,
                                             use_tc_tiling_on_sc=False),
        scratch_types=[
            pltpu.VMEM((CS,), jnp.int32),
            pltpu.VMEM((3 * CS,), jnp.float32),
            pltpu.VMEM((CS, TW), jnp.int32),
            pltpu.VMEM((3 * CS,), jnp.int32),
            pltpu.VMEM((3 * CS, TW), jnp.float32),
            pltpu.VMEM((4 * CS,), jnp.float32),
            pltpu.SemaphoreType.DMA,
            pltpu.SemaphoreType.DMA,
        ],
    )
    return kern(pix, bary, faces8, verts8)


def kernel(pix_to_face, bary_coords, faces, verts):
    b, h, w = pix_to_face.shape[:3]
    n = b * h * w
    pix = pix_to_face.reshape(n).astype(jnp.int32)
    bary_in = bary_coords.reshape(n * 3)
    faces8 = jnp.pad(faces.astype(jnp.int32), ((0, 0), (0, TW - 3)))
    verts8 = jnp.pad(verts, ((0, 0), (0, TW - 3)))
    res, vf = _sc_call(pix, bary_in, faces8, verts8, n)
    results = res.reshape(b, h, w, 4)
    vertex_faces = vf.reshape(b, h, w, 3)
    bary = bary_in.reshape(b, h, w, 3)
    return results, vertex_faces, bary


# speedup vs baseline: 12.1188x; 2.9502x over previous
import functools

import jax
import jax.numpy as jnp
from jax import lax
from jax.experimental import pallas as pl
from jax.experimental.pallas import tpu as pltpu
from jax.experimental.pallas import tpu_sc as plsc

NC = 2
NS = 16
NW = NC * NS
L = 16
CS = 2048
TW = 8


def _tile_body(n_per_tile, pix_hbm, bary_hbm, faces_hbm, verts_hbm,
               res_hbm, vf_hbm, pix_v, bary_v, vf_v, vfidx_v, rows_v, res_v,
               sem0, sem1):
    wid = lax.axis_index("s") * NC + lax.axis_index("c")
    nchunks = n_per_tile // CS
    iota = lax.iota(jnp.int32, L)
    ones = jnp.full((L,), 1.0, jnp.float32)
    zeros = jnp.full((L,), 0.0, jnp.float32)
    neg1 = jnp.full((L,), -1, jnp.int32)
    splats = [jnp.full((L,), k, jnp.int32) for k in range(4)]
    iota3k = [3 * iota + k for k in range(3)]
    iota4 = 4 * iota

    def chunk_body(ch, carry):
        base = wid * n_per_tile + ch * CS
        pltpu.sync_copy(pix_hbm.at[pl.ds(base, CS)], pix_v)
        faces_cp = pltpu.async_copy(faces_hbm.at[pix_v], vf_v, sem0)
        bary_cp = pltpu.async_copy(bary_hbm.at[pl.ds(3 * base, 3 * CS)],
                                   bary_v, sem1)
        faces_cp.wait()

        def rp(g, c2):
            p = g * L + iota
            for k in range(3):
                vk = plsc.load_gather(vf_v, [p, splats[k]])
                plsc.store_scatter(vfidx_v, [3 * g * L + iota3k[k]], vk)
            return c2

        lax.fori_loop(0, CS // L, rp, 0)
        verts_cp = pltpu.async_copy(verts_hbm.at[vfidx_v], rows_v, sem0)
        pltpu.sync_copy(vfidx_v, vf_hbm.at[pl.ds(3 * base, 3 * CS)])
        bary_cp.wait()
        verts_cp.wait()

        def grp(g, c2):
            p3 = 3 * g * L
            p4 = 4 * g * L
            pixl = pix_v[pl.ds(g * L, L)]
            alpha = jnp.where(pixl != neg1, ones, zeros)
            boff = (g >> 5) * (3 * 512) + (g & 31) * L
            b0 = bary_v[pl.ds(boff, L)]
            b1 = bary_v[pl.ds(boff + 512, L)]
            b2 = bary_v[pl.ds(boff + 1024, L)]
            for c in range(3):
                r0 = plsc.load_gather(rows_v, [p3 + iota3k[0], splats[c]])
                r1 = plsc.load_gather(rows_v, [p3 + iota3k[1], splats[c]])
                r2 = plsc.load_gather(rows_v, [p3 + iota3k[2], splats[c]])
                out_c = b0 * r0 + b1 * r1 + b2 * r2
                plsc.store_scatter(res_v, [p4 + iota4 + c], out_c)
            plsc.store_scatter(res_v, [p4 + iota4 + 3], alpha)
            return c2

        lax.fori_loop(0, CS // L, grp, 0)
        pltpu.sync_copy(res_v, res_hbm.at[pl.ds(4 * base, 4 * CS)])
        return carry

    lax.fori_loop(0, nchunks, chunk_body, 0)


@functools.partial(jax.jit, static_argnames=("n",))
def _sc_call(pix, bary, faces8, verts8, n):
    n_per_tile = n // NW
    mesh = plsc.VectorSubcoreMesh(core_axis_name="c", subcore_axis_name="s",
                                  num_cores=NC, num_subcores=NS)
    kern = pl.kernel(
        functools.partial(_tile_body, n_per_tile),
        out_type=(
            jax.ShapeDtypeStruct((n * 4,), jnp.float32),
            jax.ShapeDtypeStruct((n * 3,), jnp.int32),
        ),
        mesh=mesh,
        compiler_params=pltpu.CompilerParams(needs_layout_passes=False,
                                             use_tc_tiling_on_sc=False),
        scratch_types=[
            pltpu.VMEM((CS,), jnp.int32),
            pltpu.VMEM((3 * CS,), jnp.float32),
            pltpu.VMEM((CS, TW), jnp.int32),
            pltpu.VMEM((3 * CS,), jnp.int32),
            pltpu.VMEM((3 * CS, TW), jnp.float32),
            pltpu.VMEM((4 * CS,), jnp.float32),
            pltpu.SemaphoreType.DMA,
            pltpu.SemaphoreType.DMA,
        ],
    )
    return kern(pix, bary, faces8, verts8)


def kernel(pix_to_face, bary_coords, faces, verts):
    b, h, w = pix_to_face.shape[:3]
    n = b * h * w
    pix = pix_to_face.reshape(n).astype(jnp.int32)
    bary_in = bary_coords.transpose(0, 1, 4, 3, 2).reshape(n * 3)
    faces8 = jnp.pad(faces.astype(jnp.int32), ((0, 0), (0, TW - 3)))
    verts8 = jnp.pad(verts, ((0, 0), (0, TW - 3)))
    res, vf = _sc_call(pix, bary_in, faces8, verts8, n)
    results = res.reshape(b, h, w, 4)
    vertex_faces = vf.reshape(b, h, w, 3)
    bary = jnp.squeeze(bary_coords, axis=-2)
    return results, vertex_faces, bary


# speedup vs baseline: 38.4169x; 3.1700x over previous
import functools

import jax
import jax.numpy as jnp
from jax import lax
from jax.experimental import pallas as pl
from jax.experimental.pallas import tpu as pltpu
from jax.experimental.pallas import tpu_sc as plsc

NC = 2
NS = 16
NW = NC * NS
L = 16
CS = 2048
TW = 8


def _tile_body(n_per_tile, w, pix_hbm, bary_hbm, faces_hbm, verts_hbm,
               res_hbm, vf_hbm, bout_hbm, pix_v, bary_v, vf_v, vfidx_v,
               rows_v, res_v, sem0, sem1):
    wid = lax.axis_index("s") * NC + lax.axis_index("c")
    nchunks = n_per_tile // CS
    gpr = w // L
    iota = lax.iota(jnp.int32, L)
    ones = jnp.full((L,), 1.0, jnp.float32)
    zeros = jnp.full((L,), 0.0, jnp.float32)
    neg1 = jnp.full((L,), -1, jnp.int32)
    splats = [jnp.full((L,), k, jnp.int32) for k in range(3)]

    def chunk_body(ch, carry):
        base = wid * n_per_tile + ch * CS
        pltpu.sync_copy(pix_hbm.at[pl.ds(base, CS)], pix_v)
        faces_cp = pltpu.async_copy(faces_hbm.at[pix_v], vf_v, sem0)
        bary_cp = pltpu.async_copy(bary_hbm.at[pl.ds(3 * base, 3 * CS)],
                                   bary_v, sem1)
        faces_cp.wait()

        def rp(g, c2):
            p = g * L + iota
            voff = (g // gpr) * (3 * w) + (g % gpr) * L
            for k in range(3):
                vk = plsc.load_gather(vf_v, [p, splats[k]])
                vfidx_v[pl.ds(voff + k * w, L)] = vk
            return c2

        lax.fori_loop(0, CS // L, rp, 0)
        verts_cp = pltpu.async_copy(verts_hbm.at[vfidx_v], rows_v, sem0)
        pltpu.sync_copy(vfidx_v, vf_hbm.at[pl.ds(3 * base, 3 * CS)])
        bary_cp.wait()
        pltpu.sync_copy(bary_v, bout_hbm.at[pl.ds(3 * base, 3 * CS)])
        verts_cp.wait()

        def grp(g, c2):
            bhl = g // gpr
            wl = (g % gpr) * L
            boff = bhl * (3 * w) + wl
            roff = bhl * (4 * w) + wl
            rows0 = boff + iota
            pixl = pix_v[pl.ds(g * L, L)]
            alpha = jnp.where(pixl != neg1, ones, zeros)
            b0 = bary_v[pl.ds(boff, L)]
            b1 = bary_v[pl.ds(boff + w, L)]
            b2 = bary_v[pl.ds(boff + 2 * w, L)]
            for c in range(3):
                r0 = plsc.load_gather(rows_v, [rows0, splats[c]])
                r1 = plsc.load_gather(rows_v, [rows0 + w, splats[c]])
                r2 = plsc.load_gather(rows_v, [rows0 + 2 * w, splats[c]])
                res_v[pl.ds(roff + c * w, L)] = b0 * r0 + b1 * r1 + b2 * r2
            res_v[pl.ds(roff + 3 * w, L)] = alpha
            return c2

        lax.fori_loop(0, CS // L, grp, 0)
        pltpu.sync_copy(res_v, res_hbm.at[pl.ds(4 * base, 4 * CS)])
        return carry

    lax.fori_loop(0, nchunks, chunk_body, 0)


@functools.partial(jax.jit, static_argnames=("n", "w"))
def _sc_call(pix, bary, faces8, verts8, n, w):
    n_per_tile = n // NW
    mesh = plsc.VectorSubcoreMesh(core_axis_name="c", subcore_axis_name="s",
                                  num_cores=NC, num_subcores=NS)
    kern = pl.kernel(
        functools.partial(_tile_body, n_per_tile, w),
        out_type=(
            jax.ShapeDtypeStruct((n * 4,), jnp.float32),
            jax.ShapeDtypeStruct((n * 3,), jnp.int32),
            jax.ShapeDtypeStruct((n * 3,), jnp.float32),
        ),
        mesh=mesh,
        compiler_params=pltpu.CompilerParams(needs_layout_passes=False,
                                             use_tc_tiling_on_sc=False),
        scratch_types=[
            pltpu.VMEM((CS,), jnp.int32),
            pltpu.VMEM((3 * CS,), jnp.float32),
            pltpu.VMEM((CS, TW), jnp.int32),
            pltpu.VMEM((3 * CS,), jnp.int32),
            pltpu.VMEM((3 * CS, TW), jnp.float32),
            pltpu.VMEM((4 * CS,), jnp.float32),
            pltpu.SemaphoreType.DMA,
            pltpu.SemaphoreType.DMA,
        ],
    )
    return kern(pix, bary, faces8, verts8)


def kernel(pix_to_face, bary_coords, faces, verts):
    b, h, w = pix_to_face.shape[:3]
    n = b * h * w
    pix = pix_to_face.reshape(n).astype(jnp.int32)
    bary_in = bary_coords.transpose(0, 1, 4, 3, 2).reshape(n * 3)
    faces8 = jnp.pad(faces.astype(jnp.int32), ((0, 0), (0, TW - 3)))
    verts8 = jnp.pad(verts, ((0, 0), (0, TW - 3)))
    res, vf, bout = _sc_call(pix, bary_in, faces8, verts8, n, w)
    results = res.reshape(b, h, 4, w).transpose(0, 1, 3, 2)
    vertex_faces = vf.reshape(b, h, 3, w).transpose(0, 1, 3, 2)
    bary = bout.reshape(b, h, 3, w).transpose(0, 1, 3, 2)
    return results, vertex_faces, bary


# speedup vs baseline: 46.4218x; 1.2084x over previous
import functools

import jax
import jax.numpy as jnp
from jax import lax
from jax.experimental import pallas as pl
from jax.experimental.pallas import tpu as pltpu
from jax.experimental.pallas import tpu_sc as plsc

NC = 2
NS = 16
NW = NC * NS
L = 16
CS = 1024
TW = 8


def _tile_body(n_per_tile, w, pix_hbm, bary_hbm, faces_hbm, verts_hbm,
               res_hbm, vf_hbm, bout_hbm,
               pix_a, pix_b, bary_a, bary_b, vf_a, vf_b, vfi_a, vfi_b,
               rows_a, rows_b, res_a, res_b,
               spix_a, spix_b, sfc_a, sfc_b, sby_a, sby_b, svt_a, svt_b):
    wid = lax.axis_index("s") * NC + lax.axis_index("c")
    tile_base = wid * n_per_tile
    npairs = n_per_tile // (2 * CS)
    gpr = w // L
    iota = lax.iota(jnp.int32, L)
    ones = jnp.full((L,), 1.0, jnp.float32)
    zeros = jnp.full((L,), 0.0, jnp.float32)
    neg1 = jnp.full((L,), -1, jnp.int32)
    splats = [jnp.full((L,), k, jnp.int32) for k in range(3)]

    bufs = {
        0: (pix_a, bary_a, vf_a, vfi_a, rows_a, res_a,
            spix_a, sfc_a, sby_a, svt_a),
        1: (pix_b, bary_b, vf_b, vfi_b, rows_b, res_b,
            spix_b, sfc_b, sby_b, svt_b),
    }

    def start_pix(par, base):
        pix_v, _, _, _, _, _, spix, _, _, _ = bufs[par]
        pltpu.async_copy(pix_hbm.at[pl.ds(base, CS)], pix_v, spix)

    def phase1(par, base):
        pix_v, bary_v, vf_v, _, _, _, spix, sfc, sby, _ = bufs[par]
        pltpu.make_async_copy(pix_hbm.at[pl.ds(base, CS)], pix_v, spix).wait()
        pltpu.async_copy(faces_hbm.at[pix_v], vf_v, sfc)
        pltpu.async_copy(bary_hbm.at[pl.ds(3 * base, 3 * CS)], bary_v, sby)

    def phase2(par, base):
        pix_v, bary_v, vf_v, vfi_v, rows_v, _, _, sfc, sby, svt = bufs[par]
        pltpu.make_async_copy(faces_hbm.at[pix_v], vf_v, sfc).wait()

        def rp(g, c2):
            p = g * L + iota
            voff = (g // gpr) * (3 * w) + (g % gpr) * L
            for k in range(3):
                vk = plsc.load_gather(vf_v, [p, splats[k]])
                vfi_v[pl.ds(voff + k * w, L)] = vk
            return c2

        lax.fori_loop(0, CS // L, rp, 0)
        pltpu.async_copy(verts_hbm.at[vfi_v], rows_v, svt)
        pltpu.sync_copy(vfi_v, vf_hbm.at[pl.ds(3 * base, 3 * CS)])
        pltpu.make_async_copy(bary_hbm.at[pl.ds(3 * base, 3 * CS)], bary_v,
                              sby).wait()
        pltpu.sync_copy(bary_v, bout_hbm.at[pl.ds(3 * base, 3 * CS)])

    def phase3(par, base):
        pix_v, bary_v, _, vfi_v, rows_v, res_v, _, _, _, svt = bufs[par]
        pltpu.make_async_copy(verts_hbm.at[vfi_v], rows_v, svt).wait()

        def grp(g, c2):
            bhl = g // gpr
            wl = (g % gpr) * L
            boff = bhl * (3 * w) + wl
            roff = bhl * (4 * w) + wl
            rows0 = boff + iota
            pixl = pix_v[pl.ds(g * L, L)]
            alpha = jnp.where(pixl != neg1, ones, zeros)
            b0 = bary_v[pl.ds(boff, L)]
            b1 = bary_v[pl.ds(boff + w, L)]
            b2 = bary_v[pl.ds(boff + 2 * w, L)]
            for c in range(3):
                r0 = plsc.load_gather(rows_v, [rows0, splats[c]])
                r1 = plsc.load_gather(rows_v, [rows0 + w, splats[c]])
                r2 = plsc.load_gather(rows_v, [rows0 + 2 * w, splats[c]])
                res_v[pl.ds(roff + c * w, L)] = b0 * r0 + b1 * r1 + b2 * r2
            res_v[pl.ds(roff + 3 * w, L)] = alpha
            return c2

        lax.fori_loop(0, CS // L, grp, 0)
        pltpu.sync_copy(res_v, res_hbm.at[pl.ds(4 * base, 4 * CS)])

    start_pix(0, tile_base)

    def pair_body(i2, carry):
        base_a = tile_base + 2 * i2 * CS
        base_b = base_a + CS
        phase1(0, base_a)
        phase2(0, base_a)

        @pl.when(i2 > 0)
        def _():
            phase3(1, base_b - 2 * CS)

        start_pix(1, base_b)
        phase1(1, base_b)
        phase2(1, base_b)
        phase3(0, base_a)
        nxt = jnp.where(i2 + 1 < npairs, base_a + 2 * CS, tile_base)
        start_pix(0, nxt)
        return carry

    lax.fori_loop(0, npairs, pair_body, 0)
    phase3(1, tile_base + (2 * npairs - 1) * CS)
    pltpu.make_async_copy(pix_hbm.at[pl.ds(tile_base, CS)], pix_a,
                          spix_a).wait()


@functools.partial(jax.jit, static_argnames=("n", "w"))
def _sc_call(pix, bary, faces8, verts8, n, w):
    n_per_tile = n // NW
    mesh = plsc.VectorSubcoreMesh(core_axis_name="c", subcore_axis_name="s",
                                  num_cores=NC, num_subcores=NS)
    kern = pl.kernel(
        functools.partial(_tile_body, n_per_tile, w),
        out_type=(
            jax.ShapeDtypeStruct((n * 4,), jnp.float32),
            jax.ShapeDtypeStruct((n * 3,), jnp.int32),
            jax.ShapeDtypeStruct((n * 3,), jnp.float32),
        ),
        mesh=mesh,
        compiler_params=pltpu.CompilerParams(needs_layout_passes=False,
                                             use_tc_tiling_on_sc=False),
        scratch_types=[
            pltpu.VMEM((CS,), jnp.int32),
            pltpu.VMEM((CS,), jnp.int32),
            pltpu.VMEM((3 * CS,), jnp.float32),
            pltpu.VMEM((3 * CS,), jnp.float32),
            pltpu.VMEM((CS, TW), jnp.int32),
            pltpu.VMEM((CS, TW), jnp.int32),
            pltpu.VMEM((3 * CS,), jnp.int32),
            pltpu.VMEM((3 * CS,), jnp.int32),
            pltpu.VMEM((3 * CS, TW), jnp.float32),
            pltpu.VMEM((3 * CS, TW), jnp.float32),
            pltpu.VMEM((4 * CS,), jnp.float32),
            pltpu.VMEM((4 * CS,), jnp.float32),
            pltpu.SemaphoreType.DMA,
            pltpu.SemaphoreType.DMA,
            pltpu.SemaphoreType.DMA,
            pltpu.SemaphoreType.DMA,
            pltpu.SemaphoreType.DMA,
            pltpu.SemaphoreType.DMA,
            pltpu.SemaphoreType.DMA,
            pltpu.SemaphoreType.DMA,
        ],
    )
    return kern(pix, bary, faces8, verts8)


def kernel(pix_to_face, bary_coords, faces, verts):
    b, h, w = pix_to_face.shape[:3]
    n = b * h * w
    pix = pix_to_face.reshape(n).astype(jnp.int32)
    bary_in = bary_coords.transpose(0, 1, 4, 3, 2).reshape(n * 3)
    faces8 = jnp.pad(faces.astype(jnp.int32), ((0, 0), (0, TW - 3)))
    verts8 = jnp.pad(verts, ((0, 0), (0, TW - 3)))
    res, vf, bout = _sc_call(pix, bary_in, faces8, verts8, n, w)
    results = res.reshape(b, h, 4, w).transpose(0, 1, 3, 2)
    vertex_faces = vf.reshape(b, h, 3, w).transpose(0, 1, 3, 2)
    bary = bout.reshape(b, h, 3, w).transpose(0, 1, 3, 2)
    return results, vertex_faces, bary


# speedup vs baseline: 46.4466x; 1.0005x over previous
import functools

import jax
import jax.numpy as jnp
from jax import lax
from jax.experimental import pallas as pl
from jax.experimental.pallas import tpu as pltpu
from jax.experimental.pallas import tpu_sc as plsc

NC = 2
NS = 16
NW = NC * NS
L = 16
CS = 1024
TW = 8


def _tile_body(n_per_tile, w, pix_hbm, bary_hbm, faces_hbm, verts_hbm,
               res_hbm, vf_hbm, bout_hbm,
               pix_a, pix_b, bary_a, bary_b, vf_a, vf_b, vfi_a, vfi_b,
               rows_a, rows_b, res_a, res_b,
               spix_a, spix_b, sfc_a, sfc_b, sby_a, sby_b, svt_a, svt_b):
    wid = lax.axis_index("s") * NC + lax.axis_index("c")
    tile_base = wid * n_per_tile
    npairs = n_per_tile // (2 * CS)
    gpr = w // L
    iota = lax.iota(jnp.int32, L)
    ones = jnp.full((L,), 1.0, jnp.float32)
    zeros = jnp.full((L,), 0.0, jnp.float32)
    neg1 = jnp.full((L,), -1, jnp.int32)
    splats = [jnp.full((L,), k, jnp.int32) for k in range(3)]

    bufs = {
        0: (pix_a, bary_a, vf_a, vfi_a, rows_a, res_a,
            spix_a, sfc_a, sby_a, svt_a),
        1: (pix_b, bary_b, vf_b, vfi_b, rows_b, res_b,
            spix_b, sfc_b, sby_b, svt_b),
    }

    def start_pix(par, base):
        pix_v, _, _, _, _, _, spix, _, _, _ = bufs[par]
        pltpu.async_copy(pix_hbm.at[pl.ds(base, CS)], pix_v, spix)

    def phase1(par, base):
        pix_v, bary_v, vf_v, _, _, _, spix, sfc, sby, _ = bufs[par]
        pltpu.make_async_copy(pix_hbm.at[pl.ds(base, CS)], pix_v, spix).wait()
        pltpu.async_copy(faces_hbm.at[pix_v], vf_v, sfc)
        pltpu.async_copy(bary_hbm.at[pl.ds(3 * base, 3 * CS)], bary_v, sby)

    def phase2(par, base):
        pix_v, bary_v, vf_v, vfi_v, rows_v, _, _, sfc, sby, svt = bufs[par]
        pltpu.make_async_copy(faces_hbm.at[pix_v], vf_v, sfc).wait()

        def rp(g, c2):
            p = g * L + iota
            voff = (g // gpr) * (3 * w) + (g % gpr) * L
            for k in range(3):
                vk = plsc.load_gather(vf_v, [p, splats[k]])
                vfi_v[pl.ds(voff + k * w, L)] = vk
            return c2

        lax.fori_loop(0, CS // L, rp, 0, unroll=4)
        pltpu.async_copy(verts_hbm.at[vfi_v], rows_v, svt)
        pltpu.sync_copy(vfi_v, vf_hbm.at[pl.ds(3 * base, 3 * CS)])
        pltpu.make_async_copy(bary_hbm.at[pl.ds(3 * base, 3 * CS)], bary_v,
                              sby).wait()
        pltpu.sync_copy(bary_v, bout_hbm.at[pl.ds(3 * base, 3 * CS)])

    def phase3(par, base):
        pix_v, bary_v, _, vfi_v, rows_v, res_v, _, _, _, svt = bufs[par]
        pltpu.make_async_copy(verts_hbm.at[vfi_v], rows_v, svt).wait()

        def grp(g, c2):
            bhl = g // gpr
            wl = (g % gpr) * L
            boff = bhl * (3 * w) + wl
            roff = bhl * (4 * w) + wl
            rows0 = boff + iota
            pixl = pix_v[pl.ds(g * L, L)]
            alpha = jnp.where(pixl != neg1, ones, zeros)
            b0 = bary_v[pl.ds(boff, L)]
            b1 = bary_v[pl.ds(boff + w, L)]
            b2 = bary_v[pl.ds(boff + 2 * w, L)]
            for c in range(3):
                r0 = plsc.load_gather(rows_v, [rows0, splats[c]])
                r1 = plsc.load_gather(rows_v, [rows0 + w, splats[c]])
                r2 = plsc.load_gather(rows_v, [rows0 + 2 * w, splats[c]])
                res_v[pl.ds(roff + c * w, L)] = b0 * r0 + b1 * r1 + b2 * r2
            res_v[pl.ds(roff + 3 * w, L)] = alpha
            return c2

        lax.fori_loop(0, CS // L, grp, 0, unroll=4)
        pltpu.sync_copy(res_v, res_hbm.at[pl.ds(4 * base, 4 * CS)])

    start_pix(0, tile_base)

    def pair_body(i2, carry):
        base_a = tile_base + 2 * i2 * CS
        base_b = base_a + CS
        phase1(0, base_a)
        phase2(0, base_a)

        @pl.when(i2 > 0)
        def _():
            phase3(1, base_b - 2 * CS)

        start_pix(1, base_b)
        phase1(1, base_b)
        phase2(1, base_b)
        phase3(0, base_a)
        nxt = jnp.where(i2 + 1 < npairs, base_a + 2 * CS, tile_base)
        start_pix(0, nxt)
        return carry

    lax.fori_loop(0, npairs, pair_body, 0)
    phase3(1, tile_base + (2 * npairs - 1) * CS)
    pltpu.make_async_copy(pix_hbm.at[pl.ds(tile_base, CS)], pix_a,
                          spix_a).wait()


@functools.partial(jax.jit, static_argnames=("n", "w"))
def _sc_call(pix, bary, faces8, verts8, n, w):
    n_per_tile = n // NW
    mesh = plsc.VectorSubcoreMesh(core_axis_name="c", subcore_axis_name="s",
                                  num_cores=NC, num_subcores=NS)
    kern = pl.kernel(
        functools.partial(_tile_body, n_per_tile, w),
        out_type=(
            jax.ShapeDtypeStruct((n * 4,), jnp.float32),
            jax.ShapeDtypeStruct((n * 3,), jnp.int32),
            jax.ShapeDtypeStruct((n * 3,), jnp.float32),
        ),
        mesh=mesh,
        compiler_params=pltpu.CompilerParams(needs_layout_passes=False,
                                             use_tc_tiling_on_sc=False),
        scratch_types=[
            pltpu.VMEM((CS,), jnp.int32),
            pltpu.VMEM((CS,), jnp.int32),
            pltpu.VMEM((3 * CS,), jnp.float32),
            pltpu.VMEM((3 * CS,), jnp.float32),
            pltpu.VMEM((CS, TW), jnp.int32),
            pltpu.VMEM((CS, TW), jnp.int32),
            pltpu.VMEM((3 * CS,), jnp.int32),
            pltpu.VMEM((3 * CS,), jnp.int32),
            pltpu.VMEM((3 * CS, TW), jnp.float32),
            pltpu.VMEM((3 * CS, TW), jnp.float32),
            pltpu.VMEM((4 * CS,), jnp.float32),
            pltpu.VMEM((4 * CS,), jnp.float32),
            pltpu.SemaphoreType.DMA,
            pltpu.SemaphoreType.DMA,
            pltpu.SemaphoreType.DMA,
            pltpu.SemaphoreType.DMA,
            pltpu.SemaphoreType.DMA,
            pltpu.SemaphoreType.DMA,
            pltpu.SemaphoreType.DMA,
            pltpu.SemaphoreType.DMA,
        ],
    )
    return kern(pix, bary, faces8, verts8)


def kernel(pix_to_face, bary_coords, faces, verts):
    b, h, w = pix_to_face.shape[:3]
    n = b * h * w
    pix = pix_to_face.reshape(n).astype(jnp.int32)
    bary_in = bary_coords.transpose(0, 1, 4, 3, 2).reshape(n * 3)
    faces8 = jnp.pad(faces.astype(jnp.int32), ((0, 0), (0, TW - 3)))
    verts8 = jnp.pad(verts, ((0, 0), (0, TW - 3)))
    res, vf, bout = _sc_call(pix, bary_in, faces8, verts8, n, w)
    results = res.reshape(b, h, 4, w).transpose(0, 1, 3, 2)
    vertex_faces = vf.reshape(b, h, 3, w).transpose(0, 1, 3, 2)
    bary = bout.reshape(b, h, 3, w).transpose(0, 1, 3, 2)
    return results, vertex_faces, bary
